# Initial kernel scaffold; baseline (speedup 1.0000x reference)
import jax
import jax.numpy as jnp
from jax.experimental import pallas as pl


def kernel(images, stem_w, stem_b, l0b0c1_w, l0b0c1_b, l0b0c2_w, l0b0c2_b, l0b0c3_w, l0b0c3_b, l0b0dn_w, l0b0dn_b, l0b1c1_w, l0b1c1_b, l0b1c2_w, l0b1c2_b, l0b1c3_w, l0b1c3_b, l0b2c1_w, l0b2c1_b, l0b2c2_w, l0b2c2_b, l0b2c3_w, l0b2c3_b, l1b0c1_w, l1b0c1_b, l1b0c2_w, l1b0c2_b, l1b0c3_w, l1b0c3_b, l1b0dn_w, l1b0dn_b, l1b1c1_w, l1b1c1_b, l1b1c2_w, l1b1c2_b, l1b1c3_w, l1b1c3_b, l1b2c1_w, l1b2c1_b, l1b2c2_w, l1b2c2_b, l1b2c3_w, l1b2c3_b, l1b3c1_w, l1b3c1_b, l1b3c2_w, l1b3c2_b, l1b3c3_w, l1b3c3_b, l2b0c1_w, l2b0c1_b, l2b0c2_w, l2b0c2_b, l2b0c3_w, l2b0c3_b, l2b0dn_w, l2b0dn_b, l2b1c1_w, l2b1c1_b, l2b1c2_w, l2b1c2_b, l2b1c3_w, l2b1c3_b, l2b2c1_w, l2b2c1_b, l2b2c2_w, l2b2c2_b, l2b2c3_w, l2b2c3_b, l2b3c1_w, l2b3c1_b, l2b3c2_w, l2b3c2_b, l2b3c3_w, l2b3c3_b, l2b4c1_w, l2b4c1_b, l2b4c2_w, l2b4c2_b, l2b4c3_w, l2b4c3_b, l2b5c1_w, l2b5c1_b, l2b5c2_w, l2b5c2_b, l2b5c3_w, l2b5c3_b, l3b0c1_w, l3b0c1_b, l3b0c2_w, l3b0c2_b, l3b0c3_w, l3b0c3_b, l3b0dn_w, l3b0dn_b, l3b1c1_w, l3b1c1_b, l3b1c2_w, l3b1c2_b, l3b1c3_w, l3b1c3_b, l3b2c1_w, l3b2c1_b, l3b2c2_w, l3b2c2_b, l3b2c3_w, l3b2c3_b):
    raise NotImplementedError("write your pallas kernel here")


# speedup vs baseline: 1.5305x; 1.5305x over previous
import functools

import jax
import jax.numpy as jnp
from jax.experimental import pallas as pl
from jax.experimental.pallas import tpu as pltpu

_VMEM_LIMIT = 96 * 1024 * 1024
_BUDGET = 40 * 1024 * 1024


def _ru(x, m):
    return (x + m - 1) // m * m


def _mm_body(has_res, relu):
    if has_res:
        def body(a_ref, w_ref, b_ref, r_ref, o_ref):
            acc = jnp.dot(a_ref[...], w_ref[...],
                          preferred_element_type=jnp.float32)
            acc = acc + b_ref[...] + r_ref[...].astype(jnp.float32)
            if relu:
                acc = jnp.maximum(acc, 0.0)
            o_ref[...] = acc.astype(o_ref.dtype)
    else:
        def body(a_ref, w_ref, b_ref, o_ref):
            acc = jnp.dot(a_ref[...], w_ref[...],
                          preferred_element_type=jnp.float32)
            acc = acc + b_ref[...]
            if relu:
                acc = jnp.maximum(acc, 0.0)
            o_ref[...] = acc.astype(o_ref.dtype)
    return body


@functools.lru_cache(maxsize=None)
def _mm_call(M, K, N, tm, tn, has_res, relu, out_dtype):
    in_specs = [
        pl.BlockSpec((tm, K), lambda i, j: (i, 0)),
        pl.BlockSpec((K, tn), lambda i, j: (0, j)),
        pl.BlockSpec((1, tn), lambda i, j: (0, j)),
    ]
    if has_res:
        in_specs.append(pl.BlockSpec((tm, tn), lambda i, j: (i, j)))
    return pl.pallas_call(
        _mm_body(has_res, relu),
        out_shape=jax.ShapeDtypeStruct((M, N), out_dtype),
        grid=(pl.cdiv(M, tm), N // tn),
        in_specs=in_specs,
        out_specs=pl.BlockSpec((tm, tn), lambda i, j: (i, j)),
        compiler_params=pltpu.CompilerParams(
            dimension_semantics=("parallel", "parallel"),
            vmem_limit_bytes=_VMEM_LIMIT,
        ),
    )


def _mm(a, w, bias, res=None, relu=True, out_dtype=jnp.bfloat16):
    M, K = a.shape
    N = w.shape[1]
    tn = min(N, 512)
    osz = 4 if out_dtype == jnp.float32 else 2
    tm = 2048
    while tm > 128:
        per = 2 * (tm * K * 2 + K * tn * 2 + tm * tn * osz)
        if res is not None:
            per += 2 * tm * tn * 2
        if per <= _BUDGET:
            break
        tm //= 2
    tm = min(tm, _ru(M, 8))
    fn = _mm_call(M, K, N, tm, tn, res is not None, relu, out_dtype)
    args = (a, w, bias) if res is None else (a, w, bias, res)
    return fn(*args)


def _conv3_body(Wp, C, tm, relu):
    def body(a0_ref, a1_ref, w_ref, b_ref, o_ref):
        x = jnp.concatenate([a0_ref[...], a1_ref[...]], axis=0)
        acc = jnp.dot(x[0:tm], w_ref[0:C],
                      preferred_element_type=jnp.float32)
        for t in range(1, 9):
            off = (t // 3) * Wp + (t % 3)
            acc += jnp.dot(x[off:off + tm], w_ref[t * C:(t + 1) * C],
                           preferred_element_type=jnp.float32)
        acc = acc + b_ref[...]
        if relu:
            acc = jnp.maximum(acc, 0.0)
        o_ref[...] = acc.astype(o_ref.dtype)
    return body


@functools.lru_cache(maxsize=None)
def _conv3_call(M, C, N, Wp, tm, nb, relu, out_dtype):
    in_specs = [
        pl.BlockSpec((tm, C), lambda i: (i, 0)),
        pl.BlockSpec((tm, C), lambda i: (jnp.minimum(i + 1, nb - 1), 0)),
        pl.BlockSpec((9 * C, N), lambda i: (0, 0)),
        pl.BlockSpec((1, N), lambda i: (0, 0)),
    ]
    return pl.pallas_call(
        _conv3_body(Wp, C, tm, relu),
        out_shape=jax.ShapeDtypeStruct((M, N), out_dtype),
        grid=(nb,),
        in_specs=in_specs,
        out_specs=pl.BlockSpec((tm, N), lambda i: (i, 0)),
        compiler_params=pltpu.CompilerParams(
            dimension_semantics=("parallel",),
            vmem_limit_bytes=_VMEM_LIMIT,
        ),
    )


def _conv3x3_s1(x, w2d, bias, relu=True):
    B, H, W, C = x.shape
    N = w2d.shape[1]
    Wp = W + 2
    xp = jnp.pad(x, ((0, 0), (1, 1), (1, 1), (0, 0)))
    M = B * (H + 2) * Wp
    xf = xp.reshape(M, C)
    tm = 1024 if C <= 256 else 512
    tm = min(tm, _ru(M, 8))
    nb = pl.cdiv(M, tm)
    out = _conv3_call(M, C, N, Wp, tm, nb, relu, jnp.bfloat16)(
        xf, xf, w2d, bias)
    return out.reshape(B, H + 2, Wp, N)[:, :H, :W, :]


def _pool_w_body(tm):
    def body(a0_ref, a1_ref, o_ref):
        x = jnp.concatenate([a0_ref[...], a1_ref[...]], axis=0)
        C = o_ref.shape[1]
        m = jnp.maximum(x[0:tm, 0:C], x[0:tm, C:2 * C])
        o_ref[...] = jnp.maximum(m, x[1:tm + 1, 0:C])
    return body


def _pool_h_body(tm):
    def body(a0_ref, a1_ref, o_ref):
        x = jnp.concatenate([a0_ref[...], a1_ref[...]], axis=0)
        H = o_ref.shape[1]
        m = jnp.maximum(x[0:tm, 0:H], x[0:tm, H:2 * H])
        o_ref[...] = jnp.maximum(m, x[1:tm + 1, 0:H])
    return body


@functools.lru_cache(maxsize=None)
def _pool_call(M, Cin, tm, nb, which):
    body = _pool_w_body(tm) if which == "w" else _pool_h_body(tm)
    return pl.pallas_call(
        body,
        out_shape=jax.ShapeDtypeStruct((M, Cin // 2), jnp.bfloat16),
        grid=(nb,),
        in_specs=[
            pl.BlockSpec((tm, Cin), lambda i: (i, 0)),
            pl.BlockSpec((tm, Cin), lambda i: (jnp.minimum(i + 1, nb - 1), 0)),
        ],
        out_specs=pl.BlockSpec((tm, Cin // 2), lambda i: (i, 0)),
        compiler_params=pltpu.CompilerParams(
            dimension_semantics=("parallel",),
            vmem_limit_bytes=_VMEM_LIMIT,
        ),
    )


def _maxpool_3x3_s2(x):
    B, H, W, C = x.shape
    OH, OW = H // 2, W // 2
    hp, wp = H + 2, W + 2
    jv = wp // 2
    xp = jnp.pad(x, ((0, 0), (1, 1), (1, 1), (0, 0)),
                 constant_values=-jnp.inf)
    m1 = B * hp * jv
    x1 = xp.reshape(m1, 2 * C)
    tm1 = 512
    nb1 = pl.cdiv(m1, tm1)
    o1 = _pool_call(m1, 2 * C, tm1, nb1, "w")(x1, x1)
    iv = hp // 2
    m2 = B * iv
    lane2 = 2 * jv * C
    x2 = o1.reshape(m2, lane2)
    tm2 = 48
    nb2 = pl.cdiv(m2, tm2)
    o2 = _pool_call(m2, lane2, tm2, nb2, "h")(x2, x2)
    out = o2.reshape(B, iv, jv, C)[:, :OH, :OW, :]
    return out


def _conv_im2col(x, w2d, bias, kh, kw, stride, padding, relu=True):
    B, H, W, C = x.shape
    OH = (H + 2 * padding - kh) // stride + 1
    OW = (W + 2 * padding - kw) // stride + 1
    if padding:
        x = jnp.pad(x, ((0, 0), (padding, padding), (padding, padding),
                        (0, 0)))
    patches = [
        x[:, dy:dy + (OH - 1) * stride + 1:stride,
          dx:dx + (OW - 1) * stride + 1:stride, :]
        for dy in range(kh) for dx in range(kw)
    ]
    cols = jnp.concatenate(patches, axis=-1).reshape(B * OH * OW, kh * kw * C)
    out = _mm(cols, w2d, bias, relu=relu)
    return out.reshape(B, OH, OW, -1)


def _bottleneck(x, blk, out_dtype):
    s = blk["stride"]
    B, H, W, _ = x.shape
    c1w, c1b = blk["conv1"]
    c2w, c2b = blk["conv2"]
    c3w, c3b = blk["conv3"]
    planes = c1w.shape[1]

    y = _mm(x.reshape(B * H * W, -1), c1w, c1b)
    y = y.reshape(B, H, W, planes)
    if s == 1:
        y = _conv3x3_s1(y, c2w, c2b)
        OH, OW = H, W
    else:
        y = _conv_im2col(y, c2w, c2b, 3, 3, s, 1)
        OH, OW = H // s, W // s

    if "down" in blk:
        dw, db = blk["down"]
        xs = x[:, ::s, ::s, :] if s > 1 else x
        identity = _mm(xs.reshape(B * OH * OW, -1), dw, db, relu=False)
    else:
        identity = x.reshape(B * OH * OW, -1)

    out = _mm(y.reshape(B * OH * OW, planes), c3w, c3b,
              res=identity, relu=True, out_dtype=out_dtype)
    return out.reshape(B, OH, OW, -1)


_CFG = [(64, 3, 1), (128, 4, 2), (256, 6, 2), (512, 3, 2)]


def _forward(images, params):
    x = jnp.transpose(images, (0, 2, 3, 1)).astype(jnp.bfloat16)
    sw, sb = params["stem"]
    x = _conv_im2col(x, sw, sb, 7, 7, 2, 3, relu=True)
    x = _maxpool_3x3_s2(x)
    n_layers = len(params["layers"])
    for li, layer in enumerate(params["layers"]):
        for bi, blk in enumerate(layer):
            last = (li == n_layers - 1) and (bi == len(layer) - 1)
            x = _bottleneck(x, blk, jnp.float32 if last else jnp.bfloat16)
    B, Hf, Wf, C = x.shape
    return x.reshape(B, Hf * Wf, C)


def kernel(images, stem_w, stem_b, l0b0c1_w, l0b0c1_b, l0b0c2_w, l0b0c2_b, l0b0c3_w, l0b0c3_b, l0b0dn_w, l0b0dn_b, l0b1c1_w, l0b1c1_b, l0b1c2_w, l0b1c2_b, l0b1c3_w, l0b1c3_b, l0b2c1_w, l0b2c1_b, l0b2c2_w, l0b2c2_b, l0b2c3_w, l0b2c3_b, l1b0c1_w, l1b0c1_b, l1b0c2_w, l1b0c2_b, l1b0c3_w, l1b0c3_b, l1b0dn_w, l1b0dn_b, l1b1c1_w, l1b1c1_b, l1b1c2_w, l1b1c2_b, l1b1c3_w, l1b1c3_b, l1b2c1_w, l1b2c1_b, l1b2c2_w, l1b2c2_b, l1b2c3_w, l1b2c3_b, l1b3c1_w, l1b3c1_b, l1b3c2_w, l1b3c2_b, l1b3c3_w, l1b3c3_b, l2b0c1_w, l2b0c1_b, l2b0c2_w, l2b0c2_b, l2b0c3_w, l2b0c3_b, l2b0dn_w, l2b0dn_b, l2b1c1_w, l2b1c1_b, l2b1c2_w, l2b1c2_b, l2b1c3_w, l2b1c3_b, l2b2c1_w, l2b2c1_b, l2b2c2_w, l2b2c2_b, l2b2c3_w, l2b2c3_b, l2b3c1_w, l2b3c1_b, l2b3c2_w, l2b3c2_b, l2b3c3_w, l2b3c3_b, l2b4c1_w, l2b4c1_b, l2b4c2_w, l2b4c2_b, l2b4c3_w, l2b4c3_b, l2b5c1_w, l2b5c1_b, l2b5c2_w, l2b5c2_b, l2b5c3_w, l2b5c3_b, l3b0c1_w, l3b0c1_b, l3b0c2_w, l3b0c2_b, l3b0c3_w, l3b0c3_b, l3b0dn_w, l3b0dn_b, l3b1c1_w, l3b1c1_b, l3b1c2_w, l3b1c2_b, l3b1c3_w, l3b1c3_b, l3b2c1_w, l3b2c1_b, l3b2c2_w, l3b2c2_b, l3b2c3_w, l3b2c3_b):
    _a = dict(locals())
    params = {"stem": (stem_w, stem_b), "layers": []}
    in_ch = 64
    for li, (planes, nblocks, stride) in enumerate(_CFG):
        blocks = []
        for bi in range(nblocks):
            s = stride if bi == 0 else 1
            p = f"l{li}b{bi}"
            blk = {
                "stride": s,
                "conv1": (_a[p + "c1_w"], _a[p + "c1_b"]),
                "conv2": (_a[p + "c2_w"], _a[p + "c2_b"]),
                "conv3": (_a[p + "c3_w"], _a[p + "c3_b"]),
            }
            if s != 1 or in_ch != planes * 4:
                blk["down"] = (_a[p + "dn_w"], _a[p + "dn_b"])
            blocks.append(blk)
            in_ch = planes * 4
        params["layers"].append(blocks)
    return _forward(images, params)
